# Initial kernel scaffold; baseline (speedup 1.0000x reference)
import jax
import jax.numpy as jnp
from jax.experimental import pallas as pl


def kernel(positive_items, negative_items, shorttime, lengths, longtime, lt_lengths, emb, Wt, bt, Wi, Wh, bi, bh, Wq, bq, Wk, bk, Wv, bv, Wo, bo, W1, b1, a1, W2, b2, a2, W3, b3):
    raise NotImplementedError("write your pallas kernel here")


# speedup vs baseline: 1.4604x; 1.4604x over previous
import functools

import jax
import jax.numpy as jnp
from jax import lax
from jax.experimental import pallas as pl
from jax.experimental.pallas import tpu as pltpu
from jax.experimental.pallas import tpu_sc as plsc

_D = 16
_NH = 4
_B = 4096
_HS = 50
_HL = 200
_PK = 8
_LN = _PK * _D
_BP = _B // _PK

_NC = 2
_NS = 16
_NW = _NC * _NS
_N_IDS = (_HS + _HL + 2) * _B
_PER_W = _N_IDS // _NW
_CHUNK = 2016
_NCHUNKS = _PER_W // _CHUNK

_BBLK = 128
_RB = _BBLK // _PK


def _sc_gather(emb, ids):
    mesh = plsc.VectorSubcoreMesh(
        core_axis_name="c", subcore_axis_name="s",
        num_cores=_NC, num_subcores=_NS)

    @functools.partial(
        pl.kernel,
        mesh=mesh,
        out_type=jax.ShapeDtypeStruct((_N_IDS, _D), jnp.float32),
        scratch_types=[
            pltpu.VMEM((_CHUNK,), jnp.int32),
            pltpu.VMEM((_CHUNK, _D), jnp.float32),
            pltpu.SemaphoreType.DMA,
        ],
        compiler_params=pltpu.CompilerParams(use_tc_tiling_on_sc=False),
    )
    def gk(table_hbm, idx_hbm, out_hbm, idx_v, rows_v, sem):
        wid = lax.axis_index("s") * _NC + lax.axis_index("c")
        base = wid * _PER_W

        def body(g, carry):
            off = base + g * _CHUNK
            pltpu.sync_copy(idx_hbm.at[pl.ds(off, _CHUNK)], idx_v)
            pltpu.async_copy(table_hbm.at[idx_v], rows_v, sem).wait()
            pltpu.sync_copy(rows_v, out_hbm.at[pl.ds(off, _CHUNK)])
            return carry

        lax.fori_loop(0, _NCHUNKS, body, 0)

    return gk(emb, ids)


def _tc_body(st_ref, lt_ref, td_ref, pe_ref, ne_ref, len_ref, ltlen_ref,
             wtv, btv, wir, wiz, win, whr, whz, whn,
             bir, biz, bin_, bhr, bhz, bhn,
             wq, bq, wk, bk, wv, bv, wo, bo,
             w1a, w1b, w1c, b1, a1, w2, b2, a2, w3, b3,
             pos_out, neg_out,
             gir_s, giz_s, gin_s, outs_s):
    f32 = jnp.float32
    pe = pe_ref[...]
    ne = ne_ref[...]

    li = lax.broadcasted_iota(jnp.int32, (_LN, _LN), 0)
    lj = lax.broadcasted_iota(jnp.int32, (_LN, _LN), 1)
    G16 = (li // _D == lj // _D).astype(f32)
    G4 = (li // _NH == lj // _NH).astype(f32)
    R8 = (lax.broadcasted_iota(jnp.int32, (_PK, _LN), 0)
          == lax.broadcasted_iota(jnp.int32, (_PK, _LN), 1) // _D).astype(f32)

    st2 = st_ref[...].reshape(_HS * _RB, _LN)
    gir_s[...] = (jnp.dot(st2, wir[...], preferred_element_type=f32)
                  + bir[...]).reshape(_HS, _RB, _LN)
    giz_s[...] = (jnp.dot(st2, wiz[...], preferred_element_type=f32)
                  + biz[...]).reshape(_HS, _RB, _LN)
    gin_s[...] = (jnp.dot(st2, win[...], preferred_element_type=f32)
                  + bin_[...]).reshape(_HS, _RB, _LN)

    whr_v, whz_v, whn_v = whr[...], whz[...], whn[...]
    bhr_v, bhz_v, bhn_v = bhr[...], bhz[...], bhn[...]

    def cell(t, h):
        gr = gir_s[pl.ds(t, 1), :, :].reshape(_RB, _LN)
        gz = giz_s[pl.ds(t, 1), :, :].reshape(_RB, _LN)
        gn = gin_s[pl.ds(t, 1), :, :].reshape(_RB, _LN)
        hr = jnp.dot(h, whr_v, preferred_element_type=f32) + bhr_v
        hz = jnp.dot(h, whz_v, preferred_element_type=f32) + bhz_v
        hn = jnp.dot(h, whn_v, preferred_element_type=f32) + bhn_v
        r = jax.nn.sigmoid(gr + hr)
        z = jax.nn.sigmoid(gz + hz)
        n = jnp.tanh(gn + r * hn)
        hnew = (1.0 - z) * n + z * h
        outs_s[pl.ds(t, 1), :, :] = hnew.reshape(1, _RB, _LN)
        return hnew

    lax.fori_loop(0, _HS, cell, jnp.zeros((_RB, _LN), f32))
    outs = outs_s[...]

    Lp = jnp.dot(jnp.maximum(len_ref[...].reshape(_RB, _PK), 1.0), R8,
                 preferred_element_type=f32)
    st_tio = lax.broadcasted_iota(jnp.int32, (_HS, _RB, _LN), 0).astype(f32)
    st_mask = st_tio < Lp[None, :, :]

    def attn_pool(tgt):
        prod = (outs * tgt[None, :, :]).reshape(_HS * _RB, _LN)
        sc = (jnp.dot(prod, G16, preferred_element_type=f32)
              * 0.25).reshape(_HS, _RB, _LN)
        sc = jnp.where(st_mask, sc, -1e9)
        m = jnp.max(sc, axis=0, keepdims=True)
        e = jnp.exp(sc - m)
        w = e / jnp.sum(e, axis=0, keepdims=True)
        return jnp.sum(w * outs, axis=0)

    short_pos = attn_pool(pe)
    short_neg = attn_pool(ne)

    tdp = jnp.dot(td_ref[...].reshape(_HL * _RB, _PK), R8,
                  preferred_element_type=f32).reshape(_HL, _RB, _LN)
    lt_e = lt_ref[...] + tdp * wtv[...] + btv[...]
    lt2 = lt_e.reshape(_HL * _RB, _LN)
    kk = (jnp.dot(lt2, wk[...], preferred_element_type=f32)
          + bk[...]).reshape(_HL, _RB, _LN)
    vv = (jnp.dot(lt2, wv[...], preferred_element_type=f32)
          + bv[...]).reshape(_HL, _RB, _LN)

    LTp = jnp.dot(jnp.maximum(ltlen_ref[...].reshape(_RB, _PK), 1.0), R8,
                  preferred_element_type=f32)
    lt_tio = lax.broadcasted_iota(jnp.int32, (_HL, _RB, _LN), 0).astype(f32)
    lt_mask = lt_tio < LTp[None, :, :]

    def esu(tgt):
        q = jnp.dot(tgt, wq[...], preferred_element_type=f32) + bq[...]
        prod = (kk * q[None, :, :]).reshape(_HL * _RB, _LN)
        s = (jnp.dot(prod, G4, preferred_element_type=f32)
             * 0.5).reshape(_HL, _RB, _LN)
        s = jnp.where(lt_mask, s, -1e9)
        m = jnp.max(s, axis=0, keepdims=True)
        e = jnp.exp(s - m)
        w = e / jnp.sum(e, axis=0, keepdims=True)
        ctx = jnp.sum(w * vv, axis=0)
        return jnp.dot(ctx, wo[...], preferred_element_type=f32) + bo[...]

    long_pos = esu(pe)
    long_neg = esu(ne)

    a1v = a1[0, 0]
    a2v = a2[0, 0]

    def fcn(e_t, si, lo):
        h1 = (jnp.dot(e_t, w1a[...], preferred_element_type=f32)
              + jnp.dot(si, w1b[...], preferred_element_type=f32)
              + jnp.dot(lo, w1c[...], preferred_element_type=f32)
              + b1[...])
        h1 = jnp.maximum(h1, 0.0) + a1v * jnp.minimum(h1, 0.0)
        h2 = jnp.dot(h1, w2[...], preferred_element_type=f32) + b2[...]
        h2 = jnp.maximum(h2, 0.0) + a2v * jnp.minimum(h2, 0.0)
        return jnp.dot(h2, w3[...], preferred_element_type=f32) + b3[...]

    pos_out[...] = fcn(pe, short_pos, long_pos)
    neg_out[...] = fcn(ne, short_neg, long_neg)


_WEIGHT_SHAPES = (
    (1, _LN), (1, _LN),
    (_LN, _LN), (_LN, _LN), (_LN, _LN), (_LN, _LN), (_LN, _LN), (_LN, _LN),
    (1, _LN), (1, _LN), (1, _LN), (1, _LN), (1, _LN), (1, _LN),
    (_LN, _LN), (1, _LN), (_LN, _LN), (1, _LN),
    (_LN, _LN), (1, _LN), (_LN, _LN), (1, _LN),
    (_LN, _PK * 200), (_LN, _PK * 200), (_LN, _PK * 200), (1, _PK * 200),
    (1, 1),
    (_PK * 200, _PK * 80), (1, _PK * 80), (1, 1),
    (_PK * 80, _PK * 2), (1, _PK * 2),
)


def _tc_call():
    data_specs = [
        pl.BlockSpec((_HS, _RB, _LN), lambda i: (0, i, 0)),
        pl.BlockSpec((_HL, _RB, _LN), lambda i: (0, i, 0)),
        pl.BlockSpec((_HL, _RB, _PK), lambda i: (0, i, 0)),
        pl.BlockSpec((_RB, _LN), lambda i: (i, 0)),
        pl.BlockSpec((_RB, _LN), lambda i: (i, 0)),
        pl.BlockSpec((_RB, _PK), lambda i: (i, 0)),
        pl.BlockSpec((_RB, _PK), lambda i: (i, 0)),
    ]
    w_specs = [pl.BlockSpec(s, lambda i: (0, 0)) for s in _WEIGHT_SHAPES]
    return pl.pallas_call(
        _tc_body,
        grid=(_B // _BBLK,),
        in_specs=data_specs + w_specs,
        out_specs=[pl.BlockSpec((_RB, _PK * 2), lambda i: (i, 0))] * 2,
        out_shape=[jax.ShapeDtypeStruct((_BP, _PK * 2), jnp.float32)] * 2,
        scratch_shapes=[pltpu.VMEM((_HS, _RB, _LN), jnp.float32)] * 4,
        compiler_params=pltpu.CompilerParams(
            dimension_semantics=("arbitrary",)),
    )


def kernel(positive_items, negative_items, shorttime, lengths, longtime,
           lt_lengths, emb, Wt, bt, Wi, Wh, bi, bh, Wq, bq, Wk, bk, Wv, bv,
           Wo, bo, W1, b1, a1, W2, b2, a2, W3, b3):
    i32 = jnp.int32
    f32 = jnp.float32

    ids_st = shorttime[:, :, -1].astype(i32).T.reshape(-1)
    ids_lt = longtime[:, :, 1].astype(i32).T.reshape(-1)
    ids_pn = jnp.concatenate(
        [positive_items[:, -1], negative_items[:, -1]]).astype(i32)
    all_ids = jnp.concatenate([ids_st, ids_lt, ids_pn])

    rows = _sc_gather(emb.astype(f32), all_ids)

    n_st = _HS * _B
    n_lt = _HL * _B
    st_g = rows[:n_st].reshape(_HS, _BP, _LN)
    lt_g = rows[n_st:n_st + n_lt].reshape(_HL, _BP, _LN)
    pe = rows[n_st + n_lt:n_st + n_lt + _B].reshape(_BP, _LN)
    ne = rows[n_st + n_lt + _B:].reshape(_BP, _LN)

    td = longtime[:, :, 2].astype(f32).T.reshape(_HL, _BP, _PK)
    len2 = lengths.astype(f32).reshape(_BP, _PK)
    ltlen2 = lt_lengths.astype(f32).reshape(_BP, _PK)

    eye = jnp.eye(_PK, dtype=f32)

    def k8(w):
        return jnp.kron(eye, w.astype(f32))

    def t8(v):
        return jnp.tile(v.astype(f32).reshape(1, -1), (1, _PK))

    weights = (
        t8(Wt[:, 0]), t8(bt),
        k8(Wi[:_D].T), k8(Wi[_D:2 * _D].T), k8(Wi[2 * _D:].T),
        k8(Wh[:_D].T), k8(Wh[_D:2 * _D].T), k8(Wh[2 * _D:].T),
        t8(bi[:_D]), t8(bi[_D:2 * _D]), t8(bi[2 * _D:]),
        t8(bh[:_D]), t8(bh[_D:2 * _D]), t8(bh[2 * _D:]),
        k8(Wq.T), t8(bq), k8(Wk.T), t8(bk),
        k8(Wv.T), t8(bv), k8(Wo.T), t8(bo),
        k8(W1[:, :_D].T), k8(W1[:, _D:2 * _D].T), k8(W1[:, 2 * _D:].T),
        t8(b1), a1.astype(f32).reshape(1, 1),
        k8(W2.T), t8(b2), a2.astype(f32).reshape(1, 1),
        k8(W3.T), t8(b3),
    )

    pos_p, neg_p = _tc_call()(
        st_g, lt_g, td, pe, ne, len2, ltlen2, *weights)
    return (pos_p.reshape(_B, 2), neg_p.reshape(_B, 2))


# speedup vs baseline: 1.6137x; 1.1049x over previous
import functools

import jax
import jax.numpy as jnp
from jax import lax
from jax.experimental import pallas as pl
from jax.experimental.pallas import tpu as pltpu
from jax.experimental.pallas import tpu_sc as plsc

_D = 16
_NH = 4
_B = 4096
_HS = 50
_HL = 200
_PK = 8
_LN = _PK * _D
_BP = _B // _PK

_NC = 2
_NS = 16
_NW = _NC * _NS
_NROWS = _HS + _HL + 2
_RPT = 8

_BBLK = 128
_RB = _BBLK // _PK


def _sc_gather(emb, ids_t):
    mesh = plsc.VectorSubcoreMesh(
        core_axis_name="c", subcore_axis_name="s",
        num_cores=_NC, num_subcores=_NS)

    @functools.partial(
        pl.kernel,
        mesh=mesh,
        out_type=[
            jax.ShapeDtypeStruct((_HS, _B, _D), jnp.float32),
            jax.ShapeDtypeStruct((_HL, _B, _D), jnp.float32),
            jax.ShapeDtypeStruct((_B, _D), jnp.float32),
            jax.ShapeDtypeStruct((_B, _D), jnp.float32),
        ],
        scratch_types=[
            pltpu.VMEM((1, _B), jnp.int32),
            pltpu.VMEM((_B, _D), jnp.float32),
            pltpu.SemaphoreType.DMA,
        ],
        compiler_params=pltpu.CompilerParams(use_tc_tiling_on_sc=False),
    )
    def gk(emb_hbm, ids_hbm, st_out, lt_out, pe_out, ne_out,
           idx_v, rows_v, sem):
        wid = lax.axis_index("s") * _NC + lax.axis_index("c")

        def gather_to(dst):
            pltpu.async_copy(emb_hbm.at[idx_v.at[0]], rows_v, sem).wait()
            pltpu.sync_copy(rows_v, dst)

        def body(j, carry):
            r = wid * _RPT + j

            @pl.when(r < _NROWS)
            def _():
                pltpu.sync_copy(ids_hbm.at[pl.ds(r, 1), :], idx_v)

            @pl.when(r < _HS)
            def _():
                gather_to(st_out.at[r])

            @pl.when(jnp.logical_and(r >= _HS, r < _HS + _HL))
            def _():
                gather_to(lt_out.at[r - _HS])

            @pl.when(r == _HS + _HL)
            def _():
                gather_to(pe_out)

            @pl.when(r == _HS + _HL + 1)
            def _():
                gather_to(ne_out)

            return carry

        lax.fori_loop(0, _RPT, body, 0)

    return gk(emb, ids_t)


def _tc_body(st_ref, lt_ref, td_ref, pe_ref, ne_ref, len_ref, ltlen_ref,
             wtv, btv, wir, wiz, win, whr, whz, whn,
             bir, biz, bin_, bhr, bhz, bhn,
             wq, bq, wk, bk, wv, bv, wo, bo,
             w1a, w1b, w1c, b1, a1, w2, b2, a2, w3, b3,
             pos_out, neg_out,
             gir_s, giz_s, gin_s, outs_s):
    f32 = jnp.float32
    pe = pe_ref[...]
    ne = ne_ref[...]

    li = lax.broadcasted_iota(jnp.int32, (_LN, _LN), 0)
    lj = lax.broadcasted_iota(jnp.int32, (_LN, _LN), 1)
    G16 = (li // _D == lj // _D).astype(f32)
    G4 = (li // _NH == lj // _NH).astype(f32)
    R8 = (lax.broadcasted_iota(jnp.int32, (_PK, _LN), 0)
          == lax.broadcasted_iota(jnp.int32, (_PK, _LN), 1) // _D).astype(f32)

    st2 = st_ref[...].reshape(_HS * _RB, _LN)
    gir_s[...] = (jnp.dot(st2, wir[...], preferred_element_type=f32)
                  + bir[...]).reshape(_HS, _RB, _LN)
    giz_s[...] = (jnp.dot(st2, wiz[...], preferred_element_type=f32)
                  + biz[...]).reshape(_HS, _RB, _LN)
    gin_s[...] = (jnp.dot(st2, win[...], preferred_element_type=f32)
                  + bin_[...]).reshape(_HS, _RB, _LN)

    whr_v, whz_v, whn_v = whr[...], whz[...], whn[...]
    bhr_v, bhz_v, bhn_v = bhr[...], bhz[...], bhn[...]

    def cell(t, h):
        gr = gir_s[pl.ds(t, 1), :, :].reshape(_RB, _LN)
        gz = giz_s[pl.ds(t, 1), :, :].reshape(_RB, _LN)
        gn = gin_s[pl.ds(t, 1), :, :].reshape(_RB, _LN)
        hr = jnp.dot(h, whr_v, preferred_element_type=f32) + bhr_v
        hz = jnp.dot(h, whz_v, preferred_element_type=f32) + bhz_v
        hn = jnp.dot(h, whn_v, preferred_element_type=f32) + bhn_v
        r = jax.nn.sigmoid(gr + hr)
        z = jax.nn.sigmoid(gz + hz)
        n = jnp.tanh(gn + r * hn)
        hnew = (1.0 - z) * n + z * h
        outs_s[pl.ds(t, 1), :, :] = hnew.reshape(1, _RB, _LN)
        return hnew

    lax.fori_loop(0, _HS, cell, jnp.zeros((_RB, _LN), f32))
    outs = outs_s[...]

    Lp = jnp.dot(jnp.maximum(len_ref[...].reshape(_RB, _PK), 1.0), R8,
                 preferred_element_type=f32)
    st_tio = lax.broadcasted_iota(jnp.int32, (_HS, _RB, _LN), 0).astype(f32)
    st_mask = st_tio < Lp[None, :, :]

    def attn_pool(tgt):
        prod = (outs * tgt[None, :, :]).reshape(_HS * _RB, _LN)
        sc = (jnp.dot(prod, G16, preferred_element_type=f32)
              * 0.25).reshape(_HS, _RB, _LN)
        sc = jnp.where(st_mask, sc, -1e9)
        m = jnp.max(sc, axis=0, keepdims=True)
        e = jnp.exp(sc - m)
        w = e / jnp.sum(e, axis=0, keepdims=True)
        return jnp.sum(w * outs, axis=0)

    short_pos = attn_pool(pe)
    short_neg = attn_pool(ne)

    tdp = jnp.dot(td_ref[...].reshape(_HL * _RB, _PK), R8,
                  preferred_element_type=f32).reshape(_HL, _RB, _LN)
    lt_e = lt_ref[...] + tdp * wtv[...] + btv[...]
    lt2 = lt_e.reshape(_HL * _RB, _LN)
    kk = (jnp.dot(lt2, wk[...], preferred_element_type=f32)
          + bk[...]).reshape(_HL, _RB, _LN)
    vv = (jnp.dot(lt2, wv[...], preferred_element_type=f32)
          + bv[...]).reshape(_HL, _RB, _LN)

    LTp = jnp.dot(jnp.maximum(ltlen_ref[...].reshape(_RB, _PK), 1.0), R8,
                  preferred_element_type=f32)
    lt_tio = lax.broadcasted_iota(jnp.int32, (_HL, _RB, _LN), 0).astype(f32)
    lt_mask = lt_tio < LTp[None, :, :]

    def esu(tgt):
        q = jnp.dot(tgt, wq[...], preferred_element_type=f32) + bq[...]
        prod = (kk * q[None, :, :]).reshape(_HL * _RB, _LN)
        s = (jnp.dot(prod, G4, preferred_element_type=f32)
             * 0.5).reshape(_HL, _RB, _LN)
        s = jnp.where(lt_mask, s, -1e9)
        m = jnp.max(s, axis=0, keepdims=True)
        e = jnp.exp(s - m)
        w = e / jnp.sum(e, axis=0, keepdims=True)
        ctx = jnp.sum(w * vv, axis=0)
        return jnp.dot(ctx, wo[...], preferred_element_type=f32) + bo[...]

    long_pos = esu(pe)
    long_neg = esu(ne)

    a1v = a1[0, 0]
    a2v = a2[0, 0]

    def fcn(e_t, si, lo):
        h1 = (jnp.dot(e_t, w1a[...], preferred_element_type=f32)
              + jnp.dot(si, w1b[...], preferred_element_type=f32)
              + jnp.dot(lo, w1c[...], preferred_element_type=f32)
              + b1[...])
        h1 = jnp.maximum(h1, 0.0) + a1v * jnp.minimum(h1, 0.0)
        h2 = jnp.dot(h1, w2[...], preferred_element_type=f32) + b2[...]
        h2 = jnp.maximum(h2, 0.0) + a2v * jnp.minimum(h2, 0.0)
        return jnp.dot(h2, w3[...], preferred_element_type=f32) + b3[...]

    pos_out[...] = fcn(pe, short_pos, long_pos)
    neg_out[...] = fcn(ne, short_neg, long_neg)


_WEIGHT_SHAPES = (
    (1, _LN), (1, _LN),
    (_LN, _LN), (_LN, _LN), (_LN, _LN), (_LN, _LN), (_LN, _LN), (_LN, _LN),
    (1, _LN), (1, _LN), (1, _LN), (1, _LN), (1, _LN), (1, _LN),
    (_LN, _LN), (1, _LN), (_LN, _LN), (1, _LN),
    (_LN, _LN), (1, _LN), (_LN, _LN), (1, _LN),
    (_LN, _PK * 200), (_LN, _PK * 200), (_LN, _PK * 200), (1, _PK * 200),
    (1, 1),
    (_PK * 200, _PK * 80), (1, _PK * 80), (1, 1),
    (_PK * 80, _PK * 2), (1, _PK * 2),
)


def _tc_call():
    data_specs = [
        pl.BlockSpec((_HS, _RB, _LN), lambda i: (0, i, 0)),
        pl.BlockSpec((_HL, _RB, _LN), lambda i: (0, i, 0)),
        pl.BlockSpec((_HL, _RB, _PK), lambda i: (0, i, 0)),
        pl.BlockSpec((_RB, _LN), lambda i: (i, 0)),
        pl.BlockSpec((_RB, _LN), lambda i: (i, 0)),
        pl.BlockSpec((_RB, _PK), lambda i: (i, 0)),
        pl.BlockSpec((_RB, _PK), lambda i: (i, 0)),
    ]
    w_specs = [pl.BlockSpec(s, lambda i: (0, 0)) for s in _WEIGHT_SHAPES]
    return pl.pallas_call(
        _tc_body,
        grid=(_B // _BBLK,),
        in_specs=data_specs + w_specs,
        out_specs=[pl.BlockSpec((_RB, _PK * 2), lambda i: (i, 0))] * 2,
        out_shape=[jax.ShapeDtypeStruct((_BP, _PK * 2), jnp.float32)] * 2,
        scratch_shapes=[pltpu.VMEM((_HS, _RB, _LN), jnp.float32)] * 4,
        compiler_params=pltpu.CompilerParams(
            dimension_semantics=("arbitrary",)),
    )


def kernel(positive_items, negative_items, shorttime, lengths, longtime,
           lt_lengths, emb, Wt, bt, Wi, Wh, bi, bh, Wq, bq, Wk, bk, Wv, bv,
           Wo, bo, W1, b1, a1, W2, b2, a2, W3, b3):
    i32 = jnp.int32
    f32 = jnp.float32

    ids_t = jnp.concatenate([
        shorttime[:, :, -1].astype(i32).T,
        longtime[:, :, 1].astype(i32).T,
        positive_items[:, -1].astype(i32)[None, :],
        negative_items[:, -1].astype(i32)[None, :],
    ], axis=0)

    st_r, lt_r, pe_r, ne_r = _sc_gather(emb.astype(f32), ids_t)

    st_g = st_r.reshape(_HS, _BP, _LN)
    lt_g = lt_r.reshape(_HL, _BP, _LN)
    pe = pe_r.reshape(_BP, _LN)
    ne = ne_r.reshape(_BP, _LN)
    td = longtime[:, :, 2].astype(f32).T.reshape(_HL, _BP, _PK)
    len2 = lengths.astype(f32).reshape(_BP, _PK)
    ltlen2 = lt_lengths.astype(f32).reshape(_BP, _PK)

    eye = jnp.eye(_PK, dtype=f32)

    def k8(w):
        return jnp.kron(eye, w.astype(f32))

    def t8(v):
        return jnp.tile(v.astype(f32).reshape(1, -1), (1, _PK))

    weights = (
        t8(Wt[:, 0]), t8(bt),
        k8(Wi[:_D].T), k8(Wi[_D:2 * _D].T), k8(Wi[2 * _D:].T),
        k8(Wh[:_D].T), k8(Wh[_D:2 * _D].T), k8(Wh[2 * _D:].T),
        t8(bi[:_D]), t8(bi[_D:2 * _D]), t8(bi[2 * _D:]),
        t8(bh[:_D]), t8(bh[_D:2 * _D]), t8(bh[2 * _D:]),
        k8(Wq.T), t8(bq), k8(Wk.T), t8(bk),
        k8(Wv.T), t8(bv), k8(Wo.T), t8(bo),
        k8(W1[:, :_D].T), k8(W1[:, _D:2 * _D].T), k8(W1[:, 2 * _D:].T),
        t8(b1), a1.astype(f32).reshape(1, 1),
        k8(W2.T), t8(b2), a2.astype(f32).reshape(1, 1),
        k8(W3.T), t8(b3),
    )

    pos_p, neg_p = _tc_call()(
        st_g, lt_g, td, pe, ne, len2, ltlen2, *weights)
    return (pos_p.reshape(_B, 2), neg_p.reshape(_B, 2))


# speedup vs baseline: 1.9515x; 1.2093x over previous
import functools

import jax
import jax.numpy as jnp
from jax import lax
from jax.experimental import pallas as pl
from jax.experimental.pallas import tpu as pltpu
from jax.experimental.pallas import tpu_sc as plsc

_D = 16
_NH = 4
_B = 4096
_HS = 50
_HL = 200
_PK = 8
_LN = _PK * _D
_BP = _B // _PK

_NC = 2
_NS = 16
_NW = _NC * _NS
_NROWS = _HS + _HL + 2
_RPT = 8

_BBLK = 128
_RB = _BBLK // _PK


def _sc_gather(emb, ids_t):
    mesh = plsc.VectorSubcoreMesh(
        core_axis_name="c", subcore_axis_name="s",
        num_cores=_NC, num_subcores=_NS)

    @functools.partial(
        pl.kernel,
        mesh=mesh,
        out_type=[
            jax.ShapeDtypeStruct((_HS, _BP, _LN), jnp.float32),
            jax.ShapeDtypeStruct((_HL, _BP, _LN), jnp.float32),
            jax.ShapeDtypeStruct((_BP, _LN), jnp.float32),
            jax.ShapeDtypeStruct((_BP, _LN), jnp.float32),
        ],
        scratch_types=[
            pltpu.VMEM((1, _B), jnp.int32),
            pltpu.VMEM((_B // 2, _D), jnp.float32),
            pltpu.VMEM((_BP // 2, _LN), jnp.float32),
            pltpu.SemaphoreType.DMA,
        ],
        compiler_params=pltpu.CompilerParams(use_tc_tiling_on_sc=False),
    )
    def gk(emb_hbm, ids_hbm, st_out, lt_out, pe_out, ne_out,
           idx_v, rows_v, packed_v, sem):
        wid = lax.axis_index("s") * _NC + lax.axis_index("c")

        def gather_to(dst):
            for h in range(2):
                pltpu.async_copy(
                    emb_hbm.at[idx_v.at[0, pl.ds(h * (_B // 2), _B // 2)]],
                    rows_v, sem).wait()

                def rp(p, c):
                    for l in range(_PK):
                        packed_v[p, pl.ds(l * _D, _D)] = (
                            rows_v[p * _PK + l, :])
                    return c

                lax.fori_loop(0, _BP // 2, rp, 0)
                pltpu.sync_copy(
                    packed_v, dst.at[pl.ds(h * (_BP // 2), _BP // 2), :])

        def body(j, carry):
            r = wid * _RPT + j

            @pl.when(r < _NROWS)
            def _():
                pltpu.sync_copy(ids_hbm.at[pl.ds(r, 1), :], idx_v)

            @pl.when(r < _HS)
            def _():
                gather_to(st_out.at[r])

            @pl.when(jnp.logical_and(r >= _HS, r < _HS + _HL))
            def _():
                gather_to(lt_out.at[r - _HS])

            @pl.when(r == _HS + _HL)
            def _():
                gather_to(pe_out)

            @pl.when(r == _HS + _HL + 1)
            def _():
                gather_to(ne_out)

            return carry

        lax.fori_loop(0, _RPT, body, 0)

    return gk(emb, ids_t)


def _tc_body(st_ref, lt_ref, td_ref, pe_ref, ne_ref, len_ref, ltlen_ref,
             wtv, btv, wir, wiz, win, whr, whz, whn,
             bir, biz, bin_, bhr, bhz, bhn,
             wq, bq, wk, bk, wv, bv, wo, bo,
             w1a, w1b, w1c, b1, a1, w2, b2, a2, w3, b3,
             pos_out, neg_out,
             gir_s, giz_s, gin_s, outs_s):
    f32 = jnp.float32
    pe = pe_ref[...]
    ne = ne_ref[...]

    li = lax.broadcasted_iota(jnp.int32, (_LN, _LN), 0)
    lj = lax.broadcasted_iota(jnp.int32, (_LN, _LN), 1)
    G16 = (li // _D == lj // _D).astype(f32)
    G4 = (li // _NH == lj // _NH).astype(f32)
    R8 = (lax.broadcasted_iota(jnp.int32, (_PK, _LN), 0)
          == lax.broadcasted_iota(jnp.int32, (_PK, _LN), 1) // _D).astype(f32)

    st2 = st_ref[...].reshape(_HS * _RB, _LN)
    gir_s[...] = (jnp.dot(st2, wir[...], preferred_element_type=f32)
                  + bir[...]).reshape(_HS, _RB, _LN)
    giz_s[...] = (jnp.dot(st2, wiz[...], preferred_element_type=f32)
                  + biz[...]).reshape(_HS, _RB, _LN)
    gin_s[...] = (jnp.dot(st2, win[...], preferred_element_type=f32)
                  + bin_[...]).reshape(_HS, _RB, _LN)

    whr_v, whz_v, whn_v = whr[...], whz[...], whn[...]
    bhr_v, bhz_v, bhn_v = bhr[...], bhz[...], bhn[...]

    def cell(t, h):
        gr = gir_s[pl.ds(t, 1), :, :].reshape(_RB, _LN)
        gz = giz_s[pl.ds(t, 1), :, :].reshape(_RB, _LN)
        gn = gin_s[pl.ds(t, 1), :, :].reshape(_RB, _LN)
        hr = jnp.dot(h, whr_v, preferred_element_type=f32) + bhr_v
        hz = jnp.dot(h, whz_v, preferred_element_type=f32) + bhz_v
        hn = jnp.dot(h, whn_v, preferred_element_type=f32) + bhn_v
        r = jax.nn.sigmoid(gr + hr)
        z = jax.nn.sigmoid(gz + hz)
        n = jnp.tanh(gn + r * hn)
        hnew = (1.0 - z) * n + z * h
        outs_s[pl.ds(t, 1), :, :] = hnew.reshape(1, _RB, _LN)
        return hnew

    lax.fori_loop(0, _HS, cell, jnp.zeros((_RB, _LN), f32))
    outs = outs_s[...]

    Lp = jnp.dot(jnp.maximum(len_ref[...].reshape(_RB, _PK), 1.0), R8,
                 preferred_element_type=f32)
    st_tio = lax.broadcasted_iota(jnp.int32, (_HS, _RB, _LN), 0).astype(f32)
    st_mask = st_tio < Lp[None, :, :]

    def attn_pool(tgt):
        prod = (outs * tgt[None, :, :]).reshape(_HS * _RB, _LN)
        sc = (jnp.dot(prod, G16, preferred_element_type=f32)
              * 0.25).reshape(_HS, _RB, _LN)
        sc = jnp.where(st_mask, sc, -1e9)
        m = jnp.max(sc, axis=0, keepdims=True)
        e = jnp.exp(sc - m)
        w = e / jnp.sum(e, axis=0, keepdims=True)
        return jnp.sum(w * outs, axis=0)

    short_pos = attn_pool(pe)
    short_neg = attn_pool(ne)

    tdp = jnp.dot(td_ref[...].reshape(_HL * _RB, _PK), R8,
                  preferred_element_type=f32).reshape(_HL, _RB, _LN)
    lt_e = lt_ref[...] + tdp * wtv[...] + btv[...]
    lt2 = lt_e.reshape(_HL * _RB, _LN)
    kk = (jnp.dot(lt2, wk[...], preferred_element_type=f32)
          + bk[...]).reshape(_HL, _RB, _LN)
    vv = (jnp.dot(lt2, wv[...], preferred_element_type=f32)
          + bv[...]).reshape(_HL, _RB, _LN)

    LTp = jnp.dot(jnp.maximum(ltlen_ref[...].reshape(_RB, _PK), 1.0), R8,
                  preferred_element_type=f32)
    lt_tio = lax.broadcasted_iota(jnp.int32, (_HL, _RB, _LN), 0).astype(f32)
    lt_mask = lt_tio < LTp[None, :, :]

    def esu(tgt):
        q = jnp.dot(tgt, wq[...], preferred_element_type=f32) + bq[...]
        prod = (kk * q[None, :, :]).reshape(_HL * _RB, _LN)
        s = (jnp.dot(prod, G4, preferred_element_type=f32)
             * 0.5).reshape(_HL, _RB, _LN)
        s = jnp.where(lt_mask, s, -1e9)
        m = jnp.max(s, axis=0, keepdims=True)
        e = jnp.exp(s - m)
        w = e / jnp.sum(e, axis=0, keepdims=True)
        ctx = jnp.sum(w * vv, axis=0)
        return jnp.dot(ctx, wo[...], preferred_element_type=f32) + bo[...]

    long_pos = esu(pe)
    long_neg = esu(ne)

    a1v = a1[0, 0]
    a2v = a2[0, 0]

    def fcn(e_t, si, lo):
        h1 = (jnp.dot(e_t, w1a[...], preferred_element_type=f32)
              + jnp.dot(si, w1b[...], preferred_element_type=f32)
              + jnp.dot(lo, w1c[...], preferred_element_type=f32)
              + b1[...])
        h1 = jnp.maximum(h1, 0.0) + a1v * jnp.minimum(h1, 0.0)
        h2 = jnp.dot(h1, w2[...], preferred_element_type=f32) + b2[...]
        h2 = jnp.maximum(h2, 0.0) + a2v * jnp.minimum(h2, 0.0)
        return jnp.dot(h2, w3[...], preferred_element_type=f32) + b3[...]

    pos_out[...] = fcn(pe, short_pos, long_pos)
    neg_out[...] = fcn(ne, short_neg, long_neg)


_WEIGHT_SHAPES = (
    (1, _LN), (1, _LN),
    (_LN, _LN), (_LN, _LN), (_LN, _LN), (_LN, _LN), (_LN, _LN), (_LN, _LN),
    (1, _LN), (1, _LN), (1, _LN), (1, _LN), (1, _LN), (1, _LN),
    (_LN, _LN), (1, _LN), (_LN, _LN), (1, _LN),
    (_LN, _LN), (1, _LN), (_LN, _LN), (1, _LN),
    (_LN, _PK * 200), (_LN, _PK * 200), (_LN, _PK * 200), (1, _PK * 200),
    (1, 1),
    (_PK * 200, _PK * 80), (1, _PK * 80), (1, 1),
    (_PK * 80, _PK * 2), (1, _PK * 2),
)


def _tc_call():
    data_specs = [
        pl.BlockSpec((_HS, _RB, _LN), lambda i: (0, i, 0)),
        pl.BlockSpec((_HL, _RB, _LN), lambda i: (0, i, 0)),
        pl.BlockSpec((_HL, _RB, _PK), lambda i: (0, i, 0)),
        pl.BlockSpec((_RB, _LN), lambda i: (i, 0)),
        pl.BlockSpec((_RB, _LN), lambda i: (i, 0)),
        pl.BlockSpec((_RB, _PK), lambda i: (i, 0)),
        pl.BlockSpec((_RB, _PK), lambda i: (i, 0)),
    ]
    w_specs = [pl.BlockSpec(s, lambda i: (0, 0)) for s in _WEIGHT_SHAPES]
    return pl.pallas_call(
        _tc_body,
        grid=(_B // _BBLK,),
        in_specs=data_specs + w_specs,
        out_specs=[pl.BlockSpec((_RB, _PK * 2), lambda i: (i, 0))] * 2,
        out_shape=[jax.ShapeDtypeStruct((_BP, _PK * 2), jnp.float32)] * 2,
        scratch_shapes=[pltpu.VMEM((_HS, _RB, _LN), jnp.float32)] * 4,
        compiler_params=pltpu.CompilerParams(
            dimension_semantics=("arbitrary",)),
    )


def kernel(positive_items, negative_items, shorttime, lengths, longtime,
           lt_lengths, emb, Wt, bt, Wi, Wh, bi, bh, Wq, bq, Wk, bk, Wv, bv,
           Wo, bo, W1, b1, a1, W2, b2, a2, W3, b3):
    i32 = jnp.int32
    f32 = jnp.float32

    ids_t = jnp.concatenate([
        shorttime[:, :, -1].astype(i32).T,
        longtime[:, :, 1].astype(i32).T,
        positive_items[:, -1].astype(i32)[None, :],
        negative_items[:, -1].astype(i32)[None, :],
    ], axis=0)

    st_g, lt_g, pe, ne = _sc_gather(emb.astype(f32), ids_t)
    td = longtime[:, :, 2].astype(f32).T.reshape(_HL, _BP, _PK)
    len2 = lengths.astype(f32).reshape(_BP, _PK)
    ltlen2 = lt_lengths.astype(f32).reshape(_BP, _PK)

    eye = jnp.eye(_PK, dtype=f32)

    def k8(w):
        return jnp.kron(eye, w.astype(f32))

    def t8(v):
        return jnp.tile(v.astype(f32).reshape(1, -1), (1, _PK))

    weights = (
        t8(Wt[:, 0]), t8(bt),
        k8(Wi[:_D].T), k8(Wi[_D:2 * _D].T), k8(Wi[2 * _D:].T),
        k8(Wh[:_D].T), k8(Wh[_D:2 * _D].T), k8(Wh[2 * _D:].T),
        t8(bi[:_D]), t8(bi[_D:2 * _D]), t8(bi[2 * _D:]),
        t8(bh[:_D]), t8(bh[_D:2 * _D]), t8(bh[2 * _D:]),
        k8(Wq.T), t8(bq), k8(Wk.T), t8(bk),
        k8(Wv.T), t8(bv), k8(Wo.T), t8(bo),
        k8(W1[:, :_D].T), k8(W1[:, _D:2 * _D].T), k8(W1[:, 2 * _D:].T),
        t8(b1), a1.astype(f32).reshape(1, 1),
        k8(W2.T), t8(b2), a2.astype(f32).reshape(1, 1),
        k8(W3.T), t8(b3),
    )

    pos_p, neg_p = _tc_call()(
        st_g, lt_g, td, pe, ne, len2, ltlen2, *weights)
    return (pos_p.reshape(_B, 2), neg_p.reshape(_B, 2))
